# baseline (speedup 1.0000x reference)
import functools

import jax
import jax.numpy as jnp
from jax import lax
from jax.experimental import pallas as pl
from jax.experimental.pallas import tpu as pltpu
from jax.experimental.pallas import tpu_sc as plsc

H = 768
E = 64
BASE_K = 8
MIN_K = 4
MAX_K = 12
AUX_W = 0.01
ENT_W = 0.001

T = 512
NEG = -3.0e38

NW = 32
C = 256
L = 16


def _logits_block(x_ref, wg_ref, w1_ref, b1_ref, w2_ref, b2_ref,
                  logit_ref, kf_ref, ksum_ref, esum_ref):
    i = pl.program_id(0)
    x = x_ref[...]
    xb = x.astype(jnp.bfloat16)
    wgb = wg_ref[...].astype(jnp.bfloat16)
    w1b = w1_ref[...].astype(jnp.bfloat16)
    logits = jnp.dot(xb, wgb, preferred_element_type=jnp.float32)
    h1_pre = jnp.dot(xb, w1b, preferred_element_type=jnp.float32)
    h1_pre = h1_pre + b1_ref[...]
    h1 = h1_pre * jax.nn.sigmoid(h1_pre)
    h1b = h1.astype(jnp.bfloat16).astype(jnp.float32)
    w2b = w2_ref[...].astype(jnp.bfloat16).astype(jnp.float32)
    d_pre = jnp.sum(h1b * w2b, axis=-1, keepdims=True) + b2_ref[...]
    difficulty = jax.nn.sigmoid(d_pre)
    k_float = MIN_K + difficulty * (MAX_K - MIN_K)
    k_int = jnp.clip(jnp.round(k_float), float(MIN_K), float(MAX_K))

    logit_ref[...] = logits
    kf_ref[...] = k_int

    d1 = difficulty
    ent = d1 * jnp.log(d1 + 1e-8) + (1.0 - d1) * jnp.log(1.0 - d1 + 1e-8)
    ksum_part = jnp.sum(k_float).reshape(1, 1)
    esum_part = jnp.sum(ent).reshape(1, 1)

    @pl.when(i == 0)
    def _init():
        ksum_ref[...] = ksum_part
        esum_ref[...] = esum_part

    @pl.when(i != 0)
    def _acc():
        ksum_ref[...] += ksum_part
        esum_ref[...] += esum_part


def _tree(vs, op):
    while len(vs) > 1:
        nxt = [op(vs[i], vs[i + 1]) for i in range(0, len(vs) - 1, 2)]
        if len(vs) % 2:
            nxt.append(vs[-1])
        vs = nxt
    return vs[0]


def _make_sc_route(N):
    TW = N // NW
    G = C // L
    mesh = plsc.VectorSubcoreMesh(core_axis_name="c", subcore_axis_name="s")

    @functools.partial(
        pl.kernel, mesh=mesh,
        out_type=[jax.ShapeDtypeStruct((N * E,), jnp.float32),
                  jax.ShapeDtypeStruct((NW, E), jnp.float32)],
        scratch_types=[pltpu.VMEM((C * E,), jnp.float32),
                       pltpu.VMEM((C * E,), jnp.float32),
                       pltpu.VMEM((TW + L,), jnp.float32)],
        compiler_params=pltpu.CompilerParams(needs_layout_passes=False),
    )
    def _sc_route(logits_hbm, kf_hbm, out_hbm, tpe_hbm, inb, outb, kb):
        wid = lax.axis_index("s") * 2 + lax.axis_index("c")
        base = wid * TW
        pltpu.sync_copy(kf_hbm.at[pl.ds(base, TW)], kb.at[pl.ds(0, TW)])
        iota = lax.iota(jnp.int32, L)
        zero = jnp.zeros((L,), jnp.float32)

        def chunk_body(c, acc):
            cb = base + c * C
            pltpu.sync_copy(logits_hbm.at[pl.ds(cb * E, C * E)], inb)

            def token_body(t, acc2):
                off = t * E
                lv = [inb[pl.ds(off + j * L, L)] for j in range(4)]
                sv = [lax.sort(v) for v in lv]
                m01 = lax.sort(jnp.maximum(sv[0], lax.rev(sv[1], (0,))))
                m23 = lax.sort(jnp.maximum(sv[2], lax.rev(sv[3], (0,))))
                m = lax.sort(jnp.maximum(m01, lax.rev(m23, (0,))))
                mx = jnp.max(m)
                k = kb[pl.ds(c * C + t, L)][0].astype(jnp.int32)
                thresh = jnp.max(jnp.where(iota == (L - k), m, NEG))
                ev = [jnp.where(v >= thresh, jnp.exp(v - mx), 0.0) for v in lv]
                s_tot = jnp.sum(ev[0] + ev[1] + ev[2] + ev[3])
                s_vec = zero + s_tot
                wv = []
                for j in range(4):
                    w = ev[j] / s_vec
                    outb[pl.ds(off + j * L, L)] = w
                    wv.append(w)
                return (acc2[0] + wv[0], acc2[1] + wv[1],
                        acc2[2] + wv[2], acc2[3] + wv[3])

            acc = lax.fori_loop(0, C, token_body, acc)
            pltpu.sync_copy(outb, out_hbm.at[pl.ds(cb * E, C * E)])
            return acc

        acc = lax.fori_loop(0, TW // C, chunk_body, (zero, zero, zero, zero))

        def wr_tpe(tpeb):
            for j in range(4):
                tpeb[pl.ds(j * L, L)] = acc[j]
            pltpu.sync_copy(tpeb, tpe_hbm.at[wid])
        pl.run_scoped(wr_tpe, pltpu.VMEM((E,), jnp.float32))

    return _sc_route


def kernel(hidden_states, W_gate, W1, b1, W2, b2):
    B, S, _ = hidden_states.shape
    N = B * S
    x2d = hidden_states.reshape(N, H)
    wgT = W_gate.T
    w1T = W1.T
    Hq = W1.shape[0]
    b1r = b1.reshape(1, Hq)
    w2r = W2.reshape(1, Hq)
    b2r = b2.reshape(1, 1)

    grid = (N // T,)
    logits, kf, ksum, esum = pl.pallas_call(
        _logits_block,
        grid=grid,
        in_specs=[
            pl.BlockSpec((T, H), lambda i: (i, 0)),
            pl.BlockSpec((H, E), lambda i: (0, 0)),
            pl.BlockSpec((H, Hq), lambda i: (0, 0)),
            pl.BlockSpec((1, Hq), lambda i: (0, 0)),
            pl.BlockSpec((1, Hq), lambda i: (0, 0)),
            pl.BlockSpec((1, 1), lambda i: (0, 0)),
        ],
        out_specs=[
            pl.BlockSpec((T, E), lambda i: (i, 0)),
            pl.BlockSpec((T, 1), lambda i: (i, 0)),
            pl.BlockSpec((1, 1), lambda i: (0, 0)),
            pl.BlockSpec((1, 1), lambda i: (0, 0)),
        ],
        out_shape=[
            jax.ShapeDtypeStruct((N, E), jnp.float32),
            jax.ShapeDtypeStruct((N, 1), jnp.float32),
            jax.ShapeDtypeStruct((1, 1), jnp.float32),
            jax.ShapeDtypeStruct((1, 1), jnp.float32),
        ],
        compiler_params=pltpu.CompilerParams(
            dimension_semantics=("arbitrary",),
        ),
    )(x2d, wgT, w1T, b1r, w2r, b2r)

    wflat, tpe_w = _make_sc_route(N)(logits.reshape(N * E), kf.reshape(N))
    routing_weights = wflat.reshape(B, S, E)

    avg_k = ksum[0, 0] / N
    k_penalty = jax.nn.relu(BASE_K - avg_k) ** 2
    tpe_v = jnp.sum(tpe_w, axis=0)
    mean_tpe = jnp.mean(tpe_v)
    balance_loss = jnp.sum((tpe_v - mean_tpe) ** 2) / (E - 1) / (mean_tpe + 1e-8)
    entropy_bonus = esum[0, 0] / N
    aux_loss = AUX_W * (k_penalty + balance_loss) + ENT_W * entropy_bonus
    return routing_weights, aux_loss
